# Initial kernel scaffold; baseline (speedup 1.0000x reference)
import functools

import jax
import jax.numpy as jnp
from jax import lax
from jax.experimental import pallas as pl
from jax.experimental.pallas import tpu as pltpu
from jax.experimental.pallas import tpu_sc as plsc

USER_NUM = 25000
ITEM_NUM = 25000
N_NODES = USER_NUM + ITEM_NUM
N_EDGES = 800000
HALF = 32
N_LAYERS = 3

NC = 2
NS = 16
LANES = 16

BLK = 128
BLKS_PER_SB = 16
SB_EDGES = BLK * BLKS_PER_SB
SBS_PER_TILE = 25
BLKS_PER_TILE = BLKS_PER_SB * SBS_PER_TILE
E_PAD = BLK * BLKS_PER_TILE * NS
EB_ROWS = E_PAD // BLK

ROWS_PER_TILE = N_NODES // NS
CHUNK = 625
N_CHUNKS = ROWS_PER_TILE // CHUNK


def _scale_rows(rows_buf, vals_sb, base):
    for e in range(BLK):
        v = vals_sb[base + e]
        r0 = rows_buf[e, pl.ds(0, LANES)]
        rows_buf[e, pl.ds(0, LANES)] = r0 * v
        r1 = rows_buf[e, pl.ds(LANES, LANES)]
        rows_buf[e, pl.ds(LANES, LANES)] = r1 * v


def _sc_kernel(src_hbm, dst_hbm, vals_hbm, ego0_hbm,
               mean_hbm, ego1_hbm, ego2_hbm,
               acc, src_sb, dst_sb, vals_sb, rows_buf, buf_a, buf_b, gsem):
    c = lax.axis_index("c")
    s = lax.axis_index("s")
    row0 = s * ROWS_PER_TILE
    base_out = c * N_NODES + row0
    tile_blk0 = s * BLKS_PER_TILE

    z = jnp.zeros((LANES,), jnp.float32)

    def zrow(r, carry):
        buf_a[r, pl.ds(0, LANES)] = z
        buf_a[r, pl.ds(LANES, LANES)] = z
        return carry

    lax.fori_loop(0, CHUNK, zrow, 0)
    for k in range(N_CHUNKS):
        pltpu.sync_copy(buf_a, acc.at[pl.ds(row0 + k * CHUNK, CHUNK)])
    plsc.subcore_barrier()

    def edge_phase(tab):
        def sb_body(i, carry):
            r0 = tile_blk0 + i * BLKS_PER_SB
            pltpu.sync_copy(src_hbm.at[c].at[pl.ds(r0, BLKS_PER_SB)], src_sb)
            pltpu.sync_copy(dst_hbm.at[pl.ds(r0, BLKS_PER_SB)], dst_sb)
            pltpu.sync_copy(vals_hbm.at[pl.ds(r0 * BLK, SB_EDGES)], vals_sb)

            def blk_body(b, carry2):
                pltpu.async_copy(tab.at[src_sb.at[b]], rows_buf, gsem).wait()
                _scale_rows(rows_buf, vals_sb, b * BLK)
                pltpu.sync_copy(rows_buf, acc.at[dst_sb.at[b]], add=True)
                return carry2

            lax.fori_loop(0, BLKS_PER_SB, blk_body, 0)
            return carry

        lax.fori_loop(0, SBS_PER_TILE, sb_body, 0)

    for nxt, tab in ((ego1_hbm, ego0_hbm), (ego2_hbm, ego1_hbm)):
        edge_phase(tab)
        plsc.subcore_barrier()
        for k in range(N_CHUNKS):
            pltpu.sync_copy(acc.at[pl.ds(row0 + k * CHUNK, CHUNK)], buf_b)
            pltpu.sync_copy(buf_b, nxt.at[pl.ds(base_out + k * CHUNK, CHUNK)])
            pltpu.sync_copy(buf_a, acc.at[pl.ds(row0 + k * CHUNK, CHUNK)])
        plsc.subcore_barrier()

    edge_phase(ego2_hbm)
    plsc.subcore_barrier()

    third = jnp.float32(1.0 / 3.0)
    for k in range(N_CHUNKS):
        off = base_out + k * CHUNK
        pltpu.sync_copy(ego1_hbm.at[pl.ds(off, CHUNK)], buf_a)
        pltpu.sync_copy(ego2_hbm.at[pl.ds(off, CHUNK)], buf_b)

        def addrow(r, carry):
            a0 = buf_a[r, pl.ds(0, LANES)]
            b0 = buf_b[r, pl.ds(0, LANES)]
            buf_a[r, pl.ds(0, LANES)] = a0 + b0
            a1 = buf_a[r, pl.ds(LANES, LANES)]
            b1 = buf_b[r, pl.ds(LANES, LANES)]
            buf_a[r, pl.ds(LANES, LANES)] = a1 + b1
            return carry

        lax.fori_loop(0, CHUNK, addrow, 0)
        pltpu.sync_copy(acc.at[pl.ds(row0 + k * CHUNK, CHUNK)], buf_b)

        def finrow(r, carry):
            a0 = buf_a[r, pl.ds(0, LANES)]
            b0 = buf_b[r, pl.ds(0, LANES)]
            buf_a[r, pl.ds(0, LANES)] = (a0 + b0) * third
            a1 = buf_a[r, pl.ds(LANES, LANES)]
            b1 = buf_b[r, pl.ds(LANES, LANES)]
            buf_a[r, pl.ds(LANES, LANES)] = (a1 + b1) * third
            return carry

        lax.fori_loop(0, CHUNK, finrow, 0)
        pltpu.sync_copy(buf_a, mean_hbm.at[pl.ds(off, CHUNK)])


@jax.jit
def _propagate(src2, dst2, vals, ego0):
    mesh = plsc.VectorSubcoreMesh(core_axis_name="c", subcore_axis_name="s")
    f32 = jnp.float32
    run = pl.kernel(
        _sc_kernel,
        out_type=(
            jax.ShapeDtypeStruct((NC * N_NODES, HALF), f32),
            jax.ShapeDtypeStruct((NC * N_NODES, HALF), f32),
            jax.ShapeDtypeStruct((NC * N_NODES, HALF), f32),
        ),
        mesh=mesh,
        scratch_types=[
            pltpu.VMEM_SHARED((N_NODES, HALF), f32),
            pltpu.VMEM((BLKS_PER_SB, BLK), jnp.int32),
            pltpu.VMEM((BLKS_PER_SB, BLK), jnp.int32),
            pltpu.VMEM((SB_EDGES,), f32),
            pltpu.VMEM((BLK, HALF), f32),
            pltpu.VMEM((CHUNK, HALF), f32),
            pltpu.VMEM((CHUNK, HALF), f32),
            pltpu.SemaphoreType.DMA,
        ],
    )
    mean, _, _ = run(src2, dst2, vals, ego0)
    return mean


def kernel(user_emb, item_emb, adj_indices, adj_values):
    ego = jnp.concatenate([user_emb, item_emb], axis=0)
    ego0 = jnp.concatenate([ego[:, :HALF], ego[:, HALF:]], axis=0)
    src = adj_indices[1].astype(jnp.int32)
    dst = adj_indices[0].astype(jnp.int32)
    vals = adj_values.astype(jnp.float32)
    pad = E_PAD - N_EDGES
    src = jnp.pad(src, (0, pad))
    dst = jnp.pad(dst, (0, pad))
    vals = jnp.pad(vals, (0, pad))
    src2 = jnp.stack([src, src + N_NODES]).reshape(NC, EB_ROWS, BLK)
    dst2 = dst.reshape(EB_ROWS, BLK)

    mean = _propagate(src2, dst2, vals, ego0)
    full = jnp.concatenate([mean[:N_NODES], mean[N_NODES:]], axis=1)
    return full[:USER_NUM], full[USER_NUM:]


# speedup vs baseline: 4.0935x; 4.0935x over previous
import functools

import jax
import jax.numpy as jnp
from jax import lax
from jax.experimental import pallas as pl
from jax.experimental.pallas import tpu as pltpu
from jax.experimental.pallas import tpu_sc as plsc

USER_NUM = 25000
ITEM_NUM = 25000
N_NODES = USER_NUM + ITEM_NUM
N_EDGES = 800000
HALF = 32
N_LAYERS = 3

NC = 2
NS = 16
LANES = 16

BLK = 128
BLKS_PER_SB = 16
SB_EDGES = BLK * BLKS_PER_SB
SBS_PER_TILE = 25
BLKS_PER_TILE = BLKS_PER_SB * SBS_PER_TILE
E_PAD = BLK * BLKS_PER_TILE * NS
EB_ROWS = E_PAD // BLK

N_PAD = 51200
ROWS_PER_TILE = N_PAD // NS
CHUNK = 160
N_CHUNKS = ROWS_PER_TILE // CHUNK


def _scale_rows(rows_buf, vals_sb, base):
    for g in range(BLK // LANES):
        vv = vals_sb[pl.ds(base + g * LANES, LANES)]
        for j in range(LANES):
            e = g * LANES + j
            v = vv[j]
            r0 = rows_buf[e, pl.ds(0, LANES)]
            rows_buf[e, pl.ds(0, LANES)] = r0 * v
            r1 = rows_buf[e, pl.ds(LANES, LANES)]
            rows_buf[e, pl.ds(LANES, LANES)] = r1 * v


def _sc_kernel(src_hbm, dst_hbm, vals_hbm, ego0_hbm,
               mean_hbm, ego1_hbm, ego2_hbm,
               acc, src_sb, dst_sb, vals_sb, rows_buf, buf_a, buf_b, gsem):
    c = lax.axis_index("c")
    s = lax.axis_index("s")
    row0 = s * ROWS_PER_TILE
    base_out = c * N_PAD + row0
    tile_blk0 = s * BLKS_PER_TILE

    z = jnp.zeros((LANES,), jnp.float32)

    def zrow(r, carry):
        buf_a[r, pl.ds(0, LANES)] = z
        buf_a[r, pl.ds(LANES, LANES)] = z
        return carry

    lax.fori_loop(0, CHUNK, zrow, 0)
    for k in range(N_CHUNKS):
        pltpu.sync_copy(buf_a, acc.at[pl.ds(row0 + k * CHUNK, CHUNK)])
    plsc.subcore_barrier()

    def edge_phase(tab):
        def sb_body(i, carry):
            r0 = tile_blk0 + i * BLKS_PER_SB
            pltpu.sync_copy(src_hbm.at[c].at[pl.ds(r0, BLKS_PER_SB)], src_sb)
            pltpu.sync_copy(dst_hbm.at[pl.ds(r0, BLKS_PER_SB)], dst_sb)
            pltpu.sync_copy(vals_hbm.at[pl.ds(r0 * BLK, SB_EDGES)], vals_sb)

            def blk_body(b, carry2):
                pltpu.async_copy(tab.at[src_sb.at[b]], rows_buf, gsem).wait()
                _scale_rows(rows_buf, vals_sb, b * BLK)
                pltpu.sync_copy(rows_buf, acc.at[dst_sb.at[b]], add=True)
                return carry2

            lax.fori_loop(0, BLKS_PER_SB, blk_body, 0)
            return carry

        lax.fori_loop(0, SBS_PER_TILE, sb_body, 0)

    for nxt, tab in ((ego1_hbm, ego0_hbm), (ego2_hbm, ego1_hbm)):
        edge_phase(tab)
        plsc.subcore_barrier()
        for k in range(N_CHUNKS):
            pltpu.sync_copy(acc.at[pl.ds(row0 + k * CHUNK, CHUNK)], buf_b)
            pltpu.sync_copy(buf_b, nxt.at[pl.ds(base_out + k * CHUNK, CHUNK)])
            pltpu.sync_copy(buf_a, acc.at[pl.ds(row0 + k * CHUNK, CHUNK)])
        plsc.subcore_barrier()

    edge_phase(ego2_hbm)
    plsc.subcore_barrier()

    third = jnp.float32(1.0 / 3.0)
    for k in range(N_CHUNKS):
        off = base_out + k * CHUNK
        pltpu.sync_copy(ego1_hbm.at[pl.ds(off, CHUNK)], buf_a)
        pltpu.sync_copy(ego2_hbm.at[pl.ds(off, CHUNK)], buf_b)

        def addrow(r, carry):
            a0 = buf_a[r, pl.ds(0, LANES)]
            b0 = buf_b[r, pl.ds(0, LANES)]
            buf_a[r, pl.ds(0, LANES)] = a0 + b0
            a1 = buf_a[r, pl.ds(LANES, LANES)]
            b1 = buf_b[r, pl.ds(LANES, LANES)]
            buf_a[r, pl.ds(LANES, LANES)] = a1 + b1
            return carry

        lax.fori_loop(0, CHUNK, addrow, 0)
        pltpu.sync_copy(acc.at[pl.ds(row0 + k * CHUNK, CHUNK)], buf_b)

        def finrow(r, carry):
            a0 = buf_a[r, pl.ds(0, LANES)]
            b0 = buf_b[r, pl.ds(0, LANES)]
            buf_a[r, pl.ds(0, LANES)] = (a0 + b0) * third
            a1 = buf_a[r, pl.ds(LANES, LANES)]
            b1 = buf_b[r, pl.ds(LANES, LANES)]
            buf_a[r, pl.ds(LANES, LANES)] = (a1 + b1) * third
            return carry

        lax.fori_loop(0, CHUNK, finrow, 0)
        pltpu.sync_copy(buf_a, mean_hbm.at[pl.ds(off, CHUNK)])


@jax.jit
def _propagate(src2, dst2, vals, ego0):
    mesh = plsc.VectorSubcoreMesh(core_axis_name="c", subcore_axis_name="s")
    f32 = jnp.float32
    run = pl.kernel(
        _sc_kernel,
        out_type=(
            jax.ShapeDtypeStruct((NC * N_PAD, HALF), f32),
            jax.ShapeDtypeStruct((NC * N_PAD, HALF), f32),
            jax.ShapeDtypeStruct((NC * N_PAD, HALF), f32),
        ),
        mesh=mesh,
        compiler_params=pltpu.CompilerParams(use_tc_tiling_on_sc=False),
        scratch_types=[
            pltpu.VMEM_SHARED((N_PAD, HALF), f32),
            pltpu.VMEM((BLKS_PER_SB, BLK), jnp.int32),
            pltpu.VMEM((BLKS_PER_SB, BLK), jnp.int32),
            pltpu.VMEM((SB_EDGES,), f32),
            pltpu.VMEM((BLK, HALF), f32),
            pltpu.VMEM((CHUNK, HALF), f32),
            pltpu.VMEM((CHUNK, HALF), f32),
            pltpu.SemaphoreType.DMA,
        ],
    )
    mean, _, _ = run(src2, dst2, vals, ego0)
    return mean


def kernel(user_emb, item_emb, adj_indices, adj_values):
    ego = jnp.concatenate([user_emb, item_emb], axis=0)
    ego_pad = jnp.pad(ego, ((0, N_PAD - N_NODES), (0, 0)))
    ego0 = jnp.concatenate([ego_pad[:, :HALF], ego_pad[:, HALF:]], axis=0)
    src = adj_indices[1].astype(jnp.int32)
    dst = adj_indices[0].astype(jnp.int32)
    vals = adj_values.astype(jnp.float32)
    pad = E_PAD - N_EDGES
    src = jnp.pad(src, (0, pad))
    dst = jnp.pad(dst, (0, pad))
    vals = jnp.pad(vals, (0, pad))
    src2 = jnp.stack([src, src + N_PAD]).reshape(NC, EB_ROWS, BLK)
    dst2 = dst.reshape(EB_ROWS, BLK)

    mean = _propagate(src2, dst2, vals, ego0)
    full = jnp.concatenate([mean[:N_NODES], mean[N_PAD:N_PAD + N_NODES]], axis=1)
    return full[:USER_NUM], full[USER_NUM:]


# speedup vs baseline: 5.4231x; 1.3248x over previous
import functools

import jax
import jax.numpy as jnp
from jax import lax
from jax.experimental import pallas as pl
from jax.experimental.pallas import tpu as pltpu
from jax.experimental.pallas import tpu_sc as plsc

USER_NUM = 25000
ITEM_NUM = 25000
N_NODES = USER_NUM + ITEM_NUM
N_EDGES = 800000
HALF = 32
N_LAYERS = 3

NC = 2
NS = 16
LANES = 16

BLK = 128
BLKS_PER_SB = 16
SB_EDGES = BLK * BLKS_PER_SB
SBS_PER_TILE = 25
BLKS_PER_TILE = BLKS_PER_SB * SBS_PER_TILE
E_PAD = BLK * BLKS_PER_TILE * NS
EB_ROWS = E_PAD // BLK

N_PAD = 51200
ROWS_PER_TILE = N_PAD // NS
CHUNK = 80
N_CHUNKS = ROWS_PER_TILE // CHUNK


def _scale_rows(rows_buf, vals_sb, base):
    for g in range(BLK // LANES):
        vv = vals_sb[pl.ds(base + g * LANES, LANES)]
        for j in range(LANES):
            e = g * LANES + j
            v = vv[j]
            r0 = rows_buf[e, pl.ds(0, LANES)]
            rows_buf[e, pl.ds(0, LANES)] = r0 * v
            r1 = rows_buf[e, pl.ds(LANES, LANES)]
            rows_buf[e, pl.ds(LANES, LANES)] = r1 * v


NBUF = 4


def _sc_kernel(src_hbm, dst_hbm, vals_hbm, ego0_hbm,
               mean_hbm, ego1_hbm, ego2_hbm,
               acc, src_sb, dst_sb, vals_sb, rows_bufs, buf_a, buf_b,
               gsems, ssems):
    c = lax.axis_index("c")
    s = lax.axis_index("s")
    row0 = s * ROWS_PER_TILE
    base_out = c * N_PAD + row0
    tile_blk0 = s * BLKS_PER_TILE

    z = jnp.zeros((LANES,), jnp.float32)

    def zrow(r, carry):
        buf_a[r, pl.ds(0, LANES)] = z
        buf_a[r, pl.ds(LANES, LANES)] = z
        return carry

    lax.fori_loop(0, CHUNK, zrow, 0)
    for k in range(N_CHUNKS):
        pltpu.sync_copy(buf_a, acc.at[pl.ds(row0 + k * CHUNK, CHUNK)])
    plsc.subcore_barrier()

    def edge_phase(tab):
        def sb_body(i, carry):
            r0 = tile_blk0 + i * BLKS_PER_SB
            pltpu.sync_copy(src_hbm.at[c].at[pl.ds(r0, BLKS_PER_SB)], src_sb)
            pltpu.sync_copy(dst_hbm.at[pl.ds(r0, BLKS_PER_SB)], dst_sb)
            pltpu.sync_copy(vals_hbm.at[pl.ds(r0 * BLK, SB_EDGES)], vals_sb)

            def grp_body(g, carry2):
                b0 = g * NBUF
                gds = [
                    pltpu.async_copy(tab.at[src_sb.at[b0 + j]], rows_bufs[j],
                                     gsems[j])
                    for j in range(NBUF)
                ]
                sds = []
                for j in range(NBUF):
                    gds[j].wait()
                    _scale_rows(rows_bufs[j], vals_sb, (b0 + j) * BLK)
                    sds.append(
                        pltpu.async_copy(rows_bufs[j], acc.at[dst_sb.at[b0 + j]],
                                         ssems[j], add=True))
                for d in sds:
                    d.wait()
                return carry2

            lax.fori_loop(0, BLKS_PER_SB // NBUF, grp_body, 0)
            return carry

        lax.fori_loop(0, SBS_PER_TILE, sb_body, 0)

    for nxt, tab in ((ego1_hbm, ego0_hbm), (ego2_hbm, ego1_hbm)):
        edge_phase(tab)
        plsc.subcore_barrier()
        for k in range(N_CHUNKS):
            pltpu.sync_copy(acc.at[pl.ds(row0 + k * CHUNK, CHUNK)], buf_b)
            pltpu.sync_copy(buf_b, nxt.at[pl.ds(base_out + k * CHUNK, CHUNK)])
            pltpu.sync_copy(buf_a, acc.at[pl.ds(row0 + k * CHUNK, CHUNK)])
        plsc.subcore_barrier()

    edge_phase(ego2_hbm)
    plsc.subcore_barrier()

    third = jnp.float32(1.0 / 3.0)
    for k in range(N_CHUNKS):
        off = base_out + k * CHUNK
        pltpu.sync_copy(ego1_hbm.at[pl.ds(off, CHUNK)], buf_a)
        pltpu.sync_copy(ego2_hbm.at[pl.ds(off, CHUNK)], buf_b)

        def addrow(r, carry):
            a0 = buf_a[r, pl.ds(0, LANES)]
            b0 = buf_b[r, pl.ds(0, LANES)]
            buf_a[r, pl.ds(0, LANES)] = a0 + b0
            a1 = buf_a[r, pl.ds(LANES, LANES)]
            b1 = buf_b[r, pl.ds(LANES, LANES)]
            buf_a[r, pl.ds(LANES, LANES)] = a1 + b1
            return carry

        lax.fori_loop(0, CHUNK, addrow, 0)
        pltpu.sync_copy(acc.at[pl.ds(row0 + k * CHUNK, CHUNK)], buf_b)

        def finrow(r, carry):
            a0 = buf_a[r, pl.ds(0, LANES)]
            b0 = buf_b[r, pl.ds(0, LANES)]
            buf_a[r, pl.ds(0, LANES)] = (a0 + b0) * third
            a1 = buf_a[r, pl.ds(LANES, LANES)]
            b1 = buf_b[r, pl.ds(LANES, LANES)]
            buf_a[r, pl.ds(LANES, LANES)] = (a1 + b1) * third
            return carry

        lax.fori_loop(0, CHUNK, finrow, 0)
        pltpu.sync_copy(buf_a, mean_hbm.at[pl.ds(off, CHUNK)])


@jax.jit
def _propagate(src2, dst2, vals, ego0):
    mesh = plsc.VectorSubcoreMesh(core_axis_name="c", subcore_axis_name="s")
    f32 = jnp.float32
    run = pl.kernel(
        _sc_kernel,
        out_type=(
            jax.ShapeDtypeStruct((NC * N_PAD, HALF), f32),
            jax.ShapeDtypeStruct((NC * N_PAD, HALF), f32),
            jax.ShapeDtypeStruct((NC * N_PAD, HALF), f32),
        ),
        mesh=mesh,
        compiler_params=pltpu.CompilerParams(use_tc_tiling_on_sc=False),
        scratch_types=[
            pltpu.VMEM_SHARED((N_PAD, HALF), f32),
            pltpu.VMEM((BLKS_PER_SB, BLK), jnp.int32),
            pltpu.VMEM((BLKS_PER_SB, BLK), jnp.int32),
            pltpu.VMEM((SB_EDGES,), f32),
            [pltpu.VMEM((BLK, HALF), f32) for _ in range(NBUF)],
            pltpu.VMEM((CHUNK, HALF), f32),
            pltpu.VMEM((CHUNK, HALF), f32),
            [pltpu.SemaphoreType.DMA for _ in range(NBUF)],
            [pltpu.SemaphoreType.DMA for _ in range(NBUF)],
        ],
    )
    mean, _, _ = run(src2, dst2, vals, ego0)
    return mean


def kernel(user_emb, item_emb, adj_indices, adj_values):
    ego = jnp.concatenate([user_emb, item_emb], axis=0)
    ego_pad = jnp.pad(ego, ((0, N_PAD - N_NODES), (0, 0)))
    ego0 = jnp.concatenate([ego_pad[:, :HALF], ego_pad[:, HALF:]], axis=0)
    src = adj_indices[1].astype(jnp.int32)
    dst = adj_indices[0].astype(jnp.int32)
    vals = adj_values.astype(jnp.float32)
    pad = E_PAD - N_EDGES
    src = jnp.pad(src, (0, pad))
    dst = jnp.pad(dst, (0, pad))
    vals = jnp.pad(vals, (0, pad))
    src2 = jnp.stack([src, src + N_PAD]).reshape(NC, EB_ROWS, BLK)
    dst2 = dst.reshape(EB_ROWS, BLK)

    mean = _propagate(src2, dst2, vals, ego0)
    full = jnp.concatenate([mean[:N_NODES], mean[N_PAD:N_PAD + N_NODES]], axis=1)
    return full[:USER_NUM], full[USER_NUM:]
